# baseline (speedup 1.0000x reference)
import functools

import jax
import jax.numpy as jnp
from jax import lax
from jax.experimental import pallas as pl
from jax.experimental.pallas import tpu as pltpu
from jax.experimental.pallas import tpu_sc as plsc

NC = 2
NS = 16
NW = NC * NS
CHUNK = 128
BLK = 1024


def _deg_body(colp_hbm, ewp_hbm, deg_hbm, col_v, ew_v, zero_v, deg_sp, nchunk, npad):
    c = lax.axis_index("c")
    s = lax.axis_index("s")
    wid = c * NS + s
    rpt = npad // NS
    zv = jnp.zeros((16,), jnp.float32)
    for i in range(rpt // 16):
        zero_v[pl.ds(i * 16, 16)] = zv
    pltpu.sync_copy(zero_v, deg_sp.at[pl.ds(s * rpt, rpt)])
    plsc.subcore_barrier()
    pltpu.sync_copy(colp_hbm.at[wid], col_v)
    pltpu.sync_copy(ewp_hbm.at[wid], ew_v)

    def step(j, carry):
        pltpu.sync_copy(ew_v.at[j], deg_sp.at[col_v.at[j]], add=True)
        return carry

    lax.fori_loop(0, nchunk, step, 0)
    plsc.subcore_barrier()
    pltpu.sync_copy(deg_sp.at[pl.ds(s * rpt, rpt)],
                    deg_hbm.at[c, pl.ds(s * rpt, rpt)])


def _conv_body(g_hbm, rowf_hbm, colp_hbm, ewp_hbm, agg_hbm,
               row_v, cw_r, ew_r, rows0, rows1, agg_sp,
               sg0, sg1, ss0, ss1, si0, si1, nchunk, npad, d):
    c = lax.axis_index("c")
    s = lax.axis_index("s")
    wid = c * NS + s
    rpt = npad // NS
    bufs = (rows0, rows1)
    gsems = (sg0, sg1)
    ssems = (ss0, ss1)
    isems = (si0, si1)
    zv = jnp.zeros((16,), jnp.float32)

    def zbody(i, carry):
        for q in range(d // 16):
            rows0[i, pl.ds(q * 16, 16)] = zv
        return carry

    lax.fori_loop(0, CHUNK, zbody, 0)
    for k in range(rpt // CHUNK):
        pltpu.sync_copy(rows0, agg_sp.at[pl.ds(s * rpt + k * CHUNK, CHUNK)])
    plsc.subcore_barrier()
    pltpu.sync_copy(rowf_hbm.at[wid], row_v)

    def scale(buf, slot):
        def sbody(gi, cc):
            eww = ew_r[slot, pl.ds(gi * 16, 16)]
            base = gi * 16
            for k in range(16):
                ewb = eww.at[jnp.full((16,), k, dtype=jnp.int32)].get(
                    mode='promise_in_bounds')
                e = base + k
                for q in range(d // 16):
                    buf[e, pl.ds(q * 16, 16)] = buf[e, pl.ds(q * 16, 16)] * ewb
            return cc

        lax.fori_loop(0, CHUNK // 16, sbody, 0)

    def step(t, carry):
        j0 = 2 * t
        j1 = j0 + 1
        ic0 = pltpu.async_copy(colp_hbm.at[wid, j0], cw_r.at[0], si0)
        ie0 = pltpu.async_copy(ewp_hbm.at[wid, j0], ew_r.at[0], si0)
        ic1 = pltpu.async_copy(colp_hbm.at[wid, j1], cw_r.at[1], si1)
        ie1 = pltpu.async_copy(ewp_hbm.at[wid, j1], ew_r.at[1], si1)
        g0 = pltpu.async_copy(g_hbm.at[row_v.at[pl.ds(j0 * CHUNK, CHUNK)]],
                              rows0, sg0)
        g1 = pltpu.async_copy(g_hbm.at[row_v.at[pl.ds(j1 * CHUNK, CHUNK)]],
                              rows1, sg1)
        ic0.wait()
        ie0.wait()
        g0.wait()
        scale(rows0, 0)
        s0 = pltpu.async_copy(rows0, agg_sp.at[cw_r.at[0]], ss0, add=True)
        ic1.wait()
        ie1.wait()
        g1.wait()
        scale(rows1, 1)
        s1 = pltpu.async_copy(rows1, agg_sp.at[cw_r.at[1]], ss1, add=True)
        s0.wait()
        s1.wait()
        return carry

    lax.fori_loop(0, nchunk // 2, step, 0)
    plsc.subcore_barrier()
    for k in range(rpt // CHUNK):
        pltpu.sync_copy(agg_sp.at[pl.ds(s * rpt + k * CHUNK, CHUNK)],
                        agg_hbm.at[c, pl.ds(s * rpt + k * CHUNK, CHUNK)])


def _dinv(dp):
    deg = dp[0] + dp[1]
    return jnp.where(deg > 0, lax.rsqrt(deg), 0.0)


def _g1_body(x_ref, w_ref, dp_ref, o_ref):
    dinv = _dinv(dp_ref[...])
    h = jnp.dot(x_ref[...], w_ref[...], preferred_element_type=jnp.float32,
                precision=lax.Precision.HIGHEST)
    o_ref[...] = dinv[:, None] * h


def _gelu(x):
    return 0.5 * x * (1.0 + lax.erf(x * 0.7071067811865476))


def _ln(x, g, b):
    m = jnp.mean(x, axis=-1, keepdims=True)
    v = jnp.mean((x - m) ** 2, axis=-1, keepdims=True)
    return (x - m) / jnp.sqrt(v + 1e-5) * g + b


def _mid_body(agg_ref, dp_ref, g1_ref, b1_ref, lw_ref, lb_ref, w2_ref, o_ref):
    dinv = _dinv(dp_ref[...])
    agg = agg_ref[0] + agg_ref[1]
    pre = dinv[:, None] * (agg + g1_ref[...]) + b1_ref[...]
    act = _gelu(_ln(pre, lw_ref[...], lb_ref[...]))
    h2 = jnp.dot(act, w2_ref[...], preferred_element_type=jnp.float32,
                 precision=lax.Precision.HIGHEST)
    o_ref[...] = dinv[:, None] * h2


def _fin_body(agg_ref, dp_ref, g2_ref, b2_ref, lw_ref, lb_ref, x_ref, o_ref):
    dinv = _dinv(dp_ref[...])
    agg = agg_ref[0] + agg_ref[1]
    pre = dinv[:, None] * (agg + g2_ref[...]) + b2_ref[...]
    o_ref[...] = _gelu(_ln(pre, lw_ref[...], lb_ref[...]) + x_ref[...])


@jax.jit
def kernel(x, edge_index, edge_attr, W1, b1, ln1_w, ln1_b, W2, b2, ln2_w, ln2_b):
    n, d = x.shape
    e = edge_index.shape[1]
    npad = ((n + BLK - 1) // BLK) * BLK
    nchunk = (e + NW * CHUNK - 1) // (NW * CHUNK)
    nchunk = nchunk + (nchunk % 2)
    ep = nchunk * NW * CHUNK

    row = edge_index[0]
    col = edge_index[1]
    ew = edge_attr[:, 0]
    zpad_i = jnp.zeros((ep - e,), jnp.int32)
    rowp = jnp.concatenate([row, zpad_i]).reshape(NW, nchunk, CHUNK)
    colp = jnp.concatenate([col, zpad_i]).reshape(NW, nchunk, CHUNK)
    ewp = jnp.concatenate([ew, jnp.zeros((ep - e,), jnp.float32)]).reshape(
        NW, nchunk, CHUNK)
    rowf = rowp.reshape(NW, nchunk * CHUNK)
    xp = jnp.pad(x, ((0, npad - n), (0, 0)))

    mesh = plsc.VectorSubcoreMesh(core_axis_name="c", subcore_axis_name="s",
                                  num_cores=NC, num_subcores=NS)

    deg_part = pl.kernel(
        functools.partial(_deg_body, nchunk=nchunk, npad=npad),
        out_type=jax.ShapeDtypeStruct((NC, npad), jnp.float32),
        mesh=mesh,
        scratch_types=[
            pltpu.VMEM((nchunk, CHUNK), jnp.int32),
            pltpu.VMEM((nchunk, CHUNK), jnp.float32),
            pltpu.VMEM((npad // NS,), jnp.float32),
            pltpu.VMEM_SHARED((npad,), jnp.float32),
        ],
    )(colp, ewp)

    conv = pl.kernel(
        functools.partial(_conv_body, nchunk=nchunk, npad=npad, d=d),
        out_type=jax.ShapeDtypeStruct((NC, npad, d), jnp.float32),
        mesh=mesh,
        scratch_types=[
            pltpu.VMEM((nchunk * CHUNK,), jnp.int32),
            pltpu.VMEM((2, CHUNK), jnp.int32),
            pltpu.VMEM((2, CHUNK), jnp.float32),
            pltpu.VMEM((CHUNK, d), jnp.float32),
            pltpu.VMEM((CHUNK, d), jnp.float32),
            pltpu.VMEM_SHARED((npad, d), jnp.float32),
            pltpu.SemaphoreType.DMA,
            pltpu.SemaphoreType.DMA,
            pltpu.SemaphoreType.DMA,
            pltpu.SemaphoreType.DMA,
            pltpu.SemaphoreType.DMA,
            pltpu.SemaphoreType.DMA,
        ],
    )

    grid = npad // BLK
    b1r = b1.reshape(1, d)
    b2r = b2.reshape(1, d)
    lw1 = ln1_w.reshape(1, d)
    lb1 = ln1_b.reshape(1, d)
    lw2 = ln2_w.reshape(1, d)
    lb2 = ln2_b.reshape(1, d)

    row_spec = pl.BlockSpec((BLK, d), lambda i: (i, 0))
    w_spec = pl.BlockSpec((d, d), lambda i: (0, 0))
    dp_spec = pl.BlockSpec((NC, BLK), lambda i: (0, i))
    agg_spec = pl.BlockSpec((NC, BLK, d), lambda i: (0, i, 0))
    p_spec = pl.BlockSpec((1, d), lambda i: (0, 0))

    g1 = pl.pallas_call(
        _g1_body,
        grid=(grid,),
        in_specs=[row_spec, w_spec, dp_spec],
        out_specs=row_spec,
        out_shape=jax.ShapeDtypeStruct((npad, d), jnp.float32),
    )(xp, W1, deg_part)

    agg1 = conv(g1, rowf, colp, ewp)

    g2 = pl.pallas_call(
        _mid_body,
        grid=(grid,),
        in_specs=[agg_spec, dp_spec, row_spec, p_spec, p_spec, p_spec, w_spec],
        out_specs=row_spec,
        out_shape=jax.ShapeDtypeStruct((npad, d), jnp.float32),
    )(agg1, deg_part, g1, b1r, lw1, lb1, W2)

    agg2 = conv(g2, rowf, colp, ewp)

    out = pl.pallas_call(
        _fin_body,
        grid=(grid,),
        in_specs=[agg_spec, dp_spec, row_spec, p_spec, p_spec, p_spec, row_spec],
        out_specs=row_spec,
        out_shape=jax.ShapeDtypeStruct((npad, d), jnp.float32),
    )(agg2, deg_part, g2, b2r, lw2, lb2, xp)

    return out[:n]


# speedup vs baseline: 1.4661x; 1.4661x over previous
import functools

import jax
import jax.numpy as jnp
from jax import lax
from jax.experimental import pallas as pl
from jax.experimental.pallas import tpu as pltpu
from jax.experimental.pallas import tpu_sc as plsc

NC = 2
NS = 16
NW = NC * NS
CHUNK = 128
BLK = 1024


def _deg_body(colp_hbm, ewp_hbm, deg_hbm, col_v, ew_v, zero_v, deg_sp, nchunk, npad):
    c = lax.axis_index("c")
    s = lax.axis_index("s")
    wid = c * NS + s
    rpt = npad // NS
    zv = jnp.zeros((16,), jnp.float32)
    for i in range(rpt // 16):
        zero_v[pl.ds(i * 16, 16)] = zv
    pltpu.sync_copy(zero_v, deg_sp.at[pl.ds(s * rpt, rpt)])
    plsc.subcore_barrier()
    pltpu.sync_copy(colp_hbm.at[wid], col_v)
    pltpu.sync_copy(ewp_hbm.at[wid], ew_v)

    def step(j, carry):
        pltpu.sync_copy(ew_v.at[j], deg_sp.at[col_v.at[j]], add=True)
        return carry

    lax.fori_loop(0, nchunk, step, 0)
    plsc.subcore_barrier()
    pltpu.sync_copy(deg_sp.at[pl.ds(s * rpt, rpt)],
                    deg_hbm.at[c, pl.ds(s * rpt, rpt)])


def _conv_body(g_hbm, rowf_hbm, colp_hbm, ewf_hbm, agg_hbm,
               row_v, col_v, ew_v, rows_v, agg_sp, sg0, sg1,
               nchunk, npad, d):
    c = lax.axis_index("c")
    s = lax.axis_index("s")
    wid = c * NS + s
    rpt = npad // NS
    half = CHUNK // 2
    zv = jnp.zeros((16,), jnp.float32)

    def zbody(i, carry):
        for q in range(d // 16):
            rows_v[i, pl.ds(q * 16, 16)] = zv
        return carry

    lax.fori_loop(0, CHUNK, zbody, 0)
    for k in range(rpt // CHUNK):
        pltpu.sync_copy(rows_v, agg_sp.at[pl.ds(s * rpt + k * CHUNK, CHUNK)])
    plsc.subcore_barrier()
    pltpu.sync_copy(rowf_hbm.at[wid], row_v)
    pltpu.sync_copy(colp_hbm.at[wid], col_v)
    pltpu.sync_copy(ewf_hbm.at[wid], ew_v)

    def scale(h, j):
        base_e = j * CHUNK + h * half

        def sbody(gi, cc):
            eww = ew_v[pl.ds(base_e + gi * 16, 16)]
            base = h * half + gi * 16
            for k in range(16):
                ewb = eww.at[jnp.full((16,), k, dtype=jnp.int32)].get(
                    mode='promise_in_bounds')
                e = base + k
                for q in range(d // 16):
                    rows_v[e, pl.ds(q * 16, 16)] = (
                        rows_v[e, pl.ds(q * 16, 16)] * ewb)
            return cc

        lax.fori_loop(0, half // 16, sbody, 0)

    def step(j, carry):
        jc = j * CHUNK
        g0 = pltpu.async_copy(g_hbm.at[row_v.at[pl.ds(jc, half)]],
                              rows_v.at[pl.ds(0, half)], sg0)
        g1 = pltpu.async_copy(g_hbm.at[row_v.at[pl.ds(jc + half, half)]],
                              rows_v.at[pl.ds(half, half)], sg1)
        g0.wait()
        scale(0, j)
        g1.wait()
        scale(1, j)
        pltpu.sync_copy(rows_v, agg_sp.at[col_v.at[j]], add=True)
        return carry

    lax.fori_loop(0, nchunk, step, 0)
    plsc.subcore_barrier()
    for k in range(rpt // CHUNK):
        pltpu.sync_copy(agg_sp.at[pl.ds(s * rpt + k * CHUNK, CHUNK)],
                        agg_hbm.at[c, pl.ds(s * rpt + k * CHUNK, CHUNK)])


def _dinv(dp):
    deg = dp[0] + dp[1]
    return jnp.where(deg > 0, lax.rsqrt(deg), 0.0)


def _g1_body(x_ref, w_ref, dp_ref, o_ref):
    dinv = _dinv(dp_ref[...])
    h = jnp.dot(x_ref[...], w_ref[...], preferred_element_type=jnp.float32,
                precision=lax.Precision.HIGHEST)
    o_ref[...] = dinv[:, None] * h


def _gelu(x):
    return 0.5 * x * (1.0 + lax.erf(x * 0.7071067811865476))


def _ln(x, g, b):
    m = jnp.mean(x, axis=-1, keepdims=True)
    v = jnp.mean((x - m) ** 2, axis=-1, keepdims=True)
    return (x - m) / jnp.sqrt(v + 1e-5) * g + b


def _mid_body(agg_ref, dp_ref, g1_ref, b1_ref, lw_ref, lb_ref, w2_ref, o_ref):
    dinv = _dinv(dp_ref[...])
    agg = agg_ref[0] + agg_ref[1]
    pre = dinv[:, None] * (agg + g1_ref[...]) + b1_ref[...]
    act = _gelu(_ln(pre, lw_ref[...], lb_ref[...]))
    h2 = jnp.dot(act, w2_ref[...], preferred_element_type=jnp.float32,
                 precision=lax.Precision.HIGHEST)
    o_ref[...] = dinv[:, None] * h2


def _fin_body(agg_ref, dp_ref, g2_ref, b2_ref, lw_ref, lb_ref, x_ref, o_ref):
    dinv = _dinv(dp_ref[...])
    agg = agg_ref[0] + agg_ref[1]
    pre = dinv[:, None] * (agg + g2_ref[...]) + b2_ref[...]
    o_ref[...] = _gelu(_ln(pre, lw_ref[...], lb_ref[...]) + x_ref[...])


@jax.jit
def kernel(x, edge_index, edge_attr, W1, b1, ln1_w, ln1_b, W2, b2, ln2_w, ln2_b):
    n, d = x.shape
    e = edge_index.shape[1]
    npad = ((n + BLK - 1) // BLK) * BLK
    nchunk = (e + NW * CHUNK - 1) // (NW * CHUNK)
    ep = nchunk * NW * CHUNK

    row = edge_index[0]
    col = edge_index[1]
    ew = edge_attr[:, 0]
    zpad_i = jnp.zeros((ep - e,), jnp.int32)
    rowp = jnp.concatenate([row, zpad_i]).reshape(NW, nchunk, CHUNK)
    colp = jnp.concatenate([col, zpad_i]).reshape(NW, nchunk, CHUNK)
    ewp = jnp.concatenate([ew, jnp.zeros((ep - e,), jnp.float32)]).reshape(
        NW, nchunk, CHUNK)
    rowf = rowp.reshape(NW, nchunk * CHUNK)
    ewf = ewp.reshape(NW, nchunk * CHUNK)
    xp = jnp.pad(x, ((0, npad - n), (0, 0)))

    mesh = plsc.VectorSubcoreMesh(core_axis_name="c", subcore_axis_name="s",
                                  num_cores=NC, num_subcores=NS)

    deg_part = pl.kernel(
        functools.partial(_deg_body, nchunk=nchunk, npad=npad),
        out_type=jax.ShapeDtypeStruct((NC, npad), jnp.float32),
        mesh=mesh,
        scratch_types=[
            pltpu.VMEM((nchunk, CHUNK), jnp.int32),
            pltpu.VMEM((nchunk, CHUNK), jnp.float32),
            pltpu.VMEM((npad // NS,), jnp.float32),
            pltpu.VMEM_SHARED((npad,), jnp.float32),
        ],
    )(colp, ewp)

    conv = pl.kernel(
        functools.partial(_conv_body, nchunk=nchunk, npad=npad, d=d),
        out_type=jax.ShapeDtypeStruct((NC, npad, d), jnp.float32),
        mesh=mesh,
        scratch_types=[
            pltpu.VMEM((nchunk * CHUNK,), jnp.int32),
            pltpu.VMEM((nchunk, CHUNK), jnp.int32),
            pltpu.VMEM((nchunk * CHUNK,), jnp.float32),
            pltpu.VMEM((CHUNK, d), jnp.float32),
            pltpu.VMEM_SHARED((npad, d), jnp.float32),
            pltpu.SemaphoreType.DMA,
            pltpu.SemaphoreType.DMA,
        ],
    )

    grid = npad // BLK
    b1r = b1.reshape(1, d)
    b2r = b2.reshape(1, d)
    lw1 = ln1_w.reshape(1, d)
    lb1 = ln1_b.reshape(1, d)
    lw2 = ln2_w.reshape(1, d)
    lb2 = ln2_b.reshape(1, d)

    row_spec = pl.BlockSpec((BLK, d), lambda i: (i, 0))
    w_spec = pl.BlockSpec((d, d), lambda i: (0, 0))
    dp_spec = pl.BlockSpec((NC, BLK), lambda i: (0, i))
    agg_spec = pl.BlockSpec((NC, BLK, d), lambda i: (0, i, 0))
    p_spec = pl.BlockSpec((1, d), lambda i: (0, 0))

    g1 = pl.pallas_call(
        _g1_body,
        grid=(grid,),
        in_specs=[row_spec, w_spec, dp_spec],
        out_specs=row_spec,
        out_shape=jax.ShapeDtypeStruct((npad, d), jnp.float32),
    )(xp, W1, deg_part)

    agg1 = conv(g1, rowf, colp, ewf)

    g2 = pl.pallas_call(
        _mid_body,
        grid=(grid,),
        in_specs=[agg_spec, dp_spec, row_spec, p_spec, p_spec, p_spec, w_spec],
        out_specs=row_spec,
        out_shape=jax.ShapeDtypeStruct((npad, d), jnp.float32),
    )(agg1, deg_part, g1, b1r, lw1, lb1, W2)

    agg2 = conv(g2, rowf, colp, ewf)

    out = pl.pallas_call(
        _fin_body,
        grid=(grid,),
        in_specs=[agg_spec, dp_spec, row_spec, p_spec, p_spec, p_spec, row_spec],
        out_specs=row_spec,
        out_shape=jax.ShapeDtypeStruct((npad, d), jnp.float32),
    )(agg2, deg_part, g2, b2r, lw2, lb2, xp)

    return out[:n]


# speedup vs baseline: 1.4742x; 1.0055x over previous
import functools

import jax
import jax.numpy as jnp
from jax import lax
from jax.experimental import pallas as pl
from jax.experimental.pallas import tpu as pltpu
from jax.experimental.pallas import tpu_sc as plsc

NC = 2
NS = 16
NW = NC * NS
CHUNK = 128
BLK = 1024


def _deg_body(colp_hbm, ewp_hbm, deg_hbm, col_v, ew_v, zero_v, deg_sp, nchunk, npad):
    c = lax.axis_index("c")
    s = lax.axis_index("s")
    wid = c * NS + s
    rpt = npad // NS
    zv = jnp.zeros((16,), jnp.float32)
    for i in range(rpt // 16):
        zero_v[pl.ds(i * 16, 16)] = zv
    pltpu.sync_copy(zero_v, deg_sp.at[pl.ds(s * rpt, rpt)])
    plsc.subcore_barrier()
    pltpu.sync_copy(colp_hbm.at[wid], col_v)
    pltpu.sync_copy(ewp_hbm.at[wid], ew_v)

    def step(j, carry):
        pltpu.sync_copy(ew_v.at[j], deg_sp.at[col_v.at[j]], add=True)
        return carry

    lax.fori_loop(0, nchunk, step, 0)
    plsc.subcore_barrier()
    pltpu.sync_copy(deg_sp.at[pl.ds(s * rpt, rpt)],
                    deg_hbm.at[c, pl.ds(s * rpt, rpt)])


def _conv_body(g_hbm, rowf_hbm, colp_hbm, ewf_hbm, agg_hbm,
               row_v, col_v, ew_v, rows_v, agg_sp, sg0, sg1, sg2, sg3,
               nchunk, npad, d):
    c = lax.axis_index("c")
    s = lax.axis_index("s")
    wid = c * NS + s
    rpt = npad // NS
    gsems = (sg0, sg1, sg2, sg3)
    nq = len(gsems)
    quart = CHUNK // nq
    zv = jnp.zeros((16,), jnp.float32)

    def zbody(i, carry):
        for q in range(d // 16):
            rows_v[i, pl.ds(q * 16, 16)] = zv
        return carry

    lax.fori_loop(0, CHUNK, zbody, 0)
    for k in range(rpt // CHUNK):
        pltpu.sync_copy(rows_v, agg_sp.at[pl.ds(s * rpt + k * CHUNK, CHUNK)])
    plsc.subcore_barrier()
    pltpu.sync_copy(rowf_hbm.at[wid], row_v)
    pltpu.sync_copy(colp_hbm.at[wid], col_v)
    pltpu.sync_copy(ewf_hbm.at[wid], ew_v)

    def scale(h, j):
        base_e = j * CHUNK + h * quart

        def sbody(gi, cc):
            eww = ew_v[pl.ds(base_e + gi * 16, 16)]
            base = h * quart + gi * 16
            for k in range(16):
                ewb = eww.at[jnp.full((16,), k, dtype=jnp.int32)].get(
                    mode='promise_in_bounds')
                e = base + k
                for q in range(d // 16):
                    rows_v[e, pl.ds(q * 16, 16)] = (
                        rows_v[e, pl.ds(q * 16, 16)] * ewb)
            return cc

        lax.fori_loop(0, quart // 16, sbody, 0)

    def step(j, carry):
        jc = j * CHUNK
        descs = []
        for h in range(nq):
            descs.append(pltpu.async_copy(
                g_hbm.at[row_v.at[pl.ds(jc + h * quart, quart)]],
                rows_v.at[pl.ds(h * quart, quart)], gsems[h]))
        for h in range(nq):
            descs[h].wait()
            scale(h, j)
        pltpu.sync_copy(rows_v, agg_sp.at[col_v.at[j]], add=True)
        return carry

    lax.fori_loop(0, nchunk, step, 0)
    plsc.subcore_barrier()
    for k in range(rpt // CHUNK):
        pltpu.sync_copy(agg_sp.at[pl.ds(s * rpt + k * CHUNK, CHUNK)],
                        agg_hbm.at[c, pl.ds(s * rpt + k * CHUNK, CHUNK)])


def _dinv(dp):
    deg = dp[0] + dp[1]
    return jnp.where(deg > 0, lax.rsqrt(deg), 0.0)


def _g1_body(x_ref, w_ref, dp_ref, o_ref):
    dinv = _dinv(dp_ref[...])
    h = jnp.dot(x_ref[...], w_ref[...], preferred_element_type=jnp.float32,
                precision=lax.Precision.HIGHEST)
    o_ref[...] = dinv[:, None] * h


def _gelu(x):
    return 0.5 * x * (1.0 + lax.erf(x * 0.7071067811865476))


def _ln(x, g, b):
    m = jnp.mean(x, axis=-1, keepdims=True)
    v = jnp.mean((x - m) ** 2, axis=-1, keepdims=True)
    return (x - m) / jnp.sqrt(v + 1e-5) * g + b


def _mid_body(agg_ref, dp_ref, g1_ref, b1_ref, lw_ref, lb_ref, w2_ref, o_ref):
    dinv = _dinv(dp_ref[...])
    agg = agg_ref[0] + agg_ref[1]
    pre = dinv[:, None] * (agg + g1_ref[...]) + b1_ref[...]
    act = _gelu(_ln(pre, lw_ref[...], lb_ref[...]))
    h2 = jnp.dot(act, w2_ref[...], preferred_element_type=jnp.float32,
                 precision=lax.Precision.HIGHEST)
    o_ref[...] = dinv[:, None] * h2


def _fin_body(agg_ref, dp_ref, g2_ref, b2_ref, lw_ref, lb_ref, x_ref, o_ref):
    dinv = _dinv(dp_ref[...])
    agg = agg_ref[0] + agg_ref[1]
    pre = dinv[:, None] * (agg + g2_ref[...]) + b2_ref[...]
    o_ref[...] = _gelu(_ln(pre, lw_ref[...], lb_ref[...]) + x_ref[...])


@jax.jit
def kernel(x, edge_index, edge_attr, W1, b1, ln1_w, ln1_b, W2, b2, ln2_w, ln2_b):
    n, d = x.shape
    e = edge_index.shape[1]
    npad = ((n + BLK - 1) // BLK) * BLK
    nchunk = (e + NW * CHUNK - 1) // (NW * CHUNK)
    ep = nchunk * NW * CHUNK

    row = edge_index[0]
    col = edge_index[1]
    ew = edge_attr[:, 0]
    zpad_i = jnp.zeros((ep - e,), jnp.int32)
    rowp = jnp.concatenate([row, zpad_i]).reshape(NW, nchunk, CHUNK)
    colp = jnp.concatenate([col, zpad_i]).reshape(NW, nchunk, CHUNK)
    ewp = jnp.concatenate([ew, jnp.zeros((ep - e,), jnp.float32)]).reshape(
        NW, nchunk, CHUNK)
    rowf = rowp.reshape(NW, nchunk * CHUNK)
    ewf = ewp.reshape(NW, nchunk * CHUNK)
    xp = jnp.pad(x, ((0, npad - n), (0, 0)))

    mesh = plsc.VectorSubcoreMesh(core_axis_name="c", subcore_axis_name="s",
                                  num_cores=NC, num_subcores=NS)

    deg_part = pl.kernel(
        functools.partial(_deg_body, nchunk=nchunk, npad=npad),
        out_type=jax.ShapeDtypeStruct((NC, npad), jnp.float32),
        mesh=mesh,
        scratch_types=[
            pltpu.VMEM((nchunk, CHUNK), jnp.int32),
            pltpu.VMEM((nchunk, CHUNK), jnp.float32),
            pltpu.VMEM((npad // NS,), jnp.float32),
            pltpu.VMEM_SHARED((npad,), jnp.float32),
        ],
    )(colp, ewp)

    conv = pl.kernel(
        functools.partial(_conv_body, nchunk=nchunk, npad=npad, d=d),
        out_type=jax.ShapeDtypeStruct((NC, npad, d), jnp.float32),
        mesh=mesh,
        scratch_types=[
            pltpu.VMEM((nchunk * CHUNK,), jnp.int32),
            pltpu.VMEM((nchunk, CHUNK), jnp.int32),
            pltpu.VMEM((nchunk * CHUNK,), jnp.float32),
            pltpu.VMEM((CHUNK, d), jnp.float32),
            pltpu.VMEM_SHARED((npad, d), jnp.float32),
            pltpu.SemaphoreType.DMA,
            pltpu.SemaphoreType.DMA,
            pltpu.SemaphoreType.DMA,
            pltpu.SemaphoreType.DMA,
        ],
    )

    grid = npad // BLK
    b1r = b1.reshape(1, d)
    b2r = b2.reshape(1, d)
    lw1 = ln1_w.reshape(1, d)
    lb1 = ln1_b.reshape(1, d)
    lw2 = ln2_w.reshape(1, d)
    lb2 = ln2_b.reshape(1, d)

    row_spec = pl.BlockSpec((BLK, d), lambda i: (i, 0))
    w_spec = pl.BlockSpec((d, d), lambda i: (0, 0))
    dp_spec = pl.BlockSpec((NC, BLK), lambda i: (0, i))
    agg_spec = pl.BlockSpec((NC, BLK, d), lambda i: (0, i, 0))
    p_spec = pl.BlockSpec((1, d), lambda i: (0, 0))

    g1 = pl.pallas_call(
        _g1_body,
        grid=(grid,),
        in_specs=[row_spec, w_spec, dp_spec],
        out_specs=row_spec,
        out_shape=jax.ShapeDtypeStruct((npad, d), jnp.float32),
    )(xp, W1, deg_part)

    agg1 = conv(g1, rowf, colp, ewf)

    g2 = pl.pallas_call(
        _mid_body,
        grid=(grid,),
        in_specs=[agg_spec, dp_spec, row_spec, p_spec, p_spec, p_spec, w_spec],
        out_specs=row_spec,
        out_shape=jax.ShapeDtypeStruct((npad, d), jnp.float32),
    )(agg1, deg_part, g1, b1r, lw1, lb1, W2)

    agg2 = conv(g2, rowf, colp, ewf)

    out = pl.pallas_call(
        _fin_body,
        grid=(grid,),
        in_specs=[agg_spec, dp_spec, row_spec, p_spec, p_spec, p_spec, row_spec],
        out_specs=row_spec,
        out_shape=jax.ShapeDtypeStruct((npad, d), jnp.float32),
    )(agg2, deg_part, g2, b2r, lw2, lb2, xp)

    return out[:n]
